# Initial kernel scaffold; baseline (speedup 1.0000x reference)
import jax
import jax.numpy as jnp
from jax.experimental import pallas as pl


def kernel(scores, k):
    raise NotImplementedError("write your pallas kernel here")


# speedup vs baseline: 15.2803x; 15.2803x over previous
import jax
import jax.numpy as jnp
from jax import lax
from jax.experimental import pallas as pl
from jax.experimental.pallas import tpu as pltpu

_ROWS = 64
_COLS = 32768
_BLOCK_ROWS = 8


def _topk_mask_body(k_ref, x_ref, o_ref):
    x = x_ref[...]
    xi = lax.bitcast_convert_type(x, jnp.int32)
    keys = jnp.where(xi >= 0, xi, xi ^ jnp.int32(0x7FFFFFFF))
    k = k_ref[0]

    def body(_, carry):
        lo, hi = carry
        x_and = lo & hi
        x_xor = lo ^ hi
        mid = x_and + (x_xor >> 1) + (x_xor & 1)
        cnt = jnp.sum((keys >= mid).astype(jnp.int32), axis=-1, keepdims=True)
        pred = cnt >= k
        lo = jnp.where(pred, mid, lo)
        hi = jnp.where(pred, hi, mid - jnp.int32(1))
        return lo, hi

    lo0 = jnp.full((_BLOCK_ROWS, 1), jnp.int32(-2147483648))
    hi0 = jnp.full((_BLOCK_ROWS, 1), jnp.int32(2147483647))
    lo, _ = lax.fori_loop(0, 32, body, (lo0, hi0))
    ti = jnp.where(lo >= 0, lo, lo ^ jnp.int32(0x7FFFFFFF))
    thres = lax.bitcast_convert_type(ti, jnp.float32)
    o_ref[...] = (x >= thres).astype(jnp.int32)


def kernel(scores, k):
    k_arr = jnp.reshape(jnp.asarray(k, jnp.int32), (1,))
    grid = (_ROWS // _BLOCK_ROWS,)
    return pl.pallas_call(
        _topk_mask_body,
        grid=grid,
        in_specs=[
            pl.BlockSpec(memory_space=pltpu.SMEM),
            pl.BlockSpec((_BLOCK_ROWS, _COLS), lambda i: (i, 0)),
        ],
        out_specs=pl.BlockSpec((_BLOCK_ROWS, _COLS), lambda i: (i, 0)),
        out_shape=jax.ShapeDtypeStruct((_ROWS, _COLS), jnp.int32),
    )(k_arr, scores)


# speedup vs baseline: 33.4405x; 2.1885x over previous
import jax
import jax.numpy as jnp
from jax import lax
from jax.experimental import pallas as pl
from jax.experimental.pallas import tpu as pltpu

_ROWS = 64
_COLS = 32768
_BLOCK_ROWS = 64


def _topk_mask_body(k_ref, x_ref, o_ref):
    x = x_ref[...]
    xi = lax.bitcast_convert_type(x, jnp.int32)
    keys = jnp.where(xi >= 0, xi, xi ^ jnp.int32(0x7FFFFFFF))
    k = k_ref[0]

    def body(_, carry):
        lo, hi = carry
        x_and = lo & hi
        x_xor = lo ^ hi
        mid = x_and + (x_xor >> 1) + (x_xor & 1)
        cnt = jnp.sum((keys >= mid).astype(jnp.int32), axis=-1, keepdims=True)
        pred = cnt >= k
        lo = jnp.where(pred, mid, lo)
        hi = jnp.where(pred, hi, mid - jnp.int32(1))
        return lo, hi

    lo0 = jnp.full((_BLOCK_ROWS, 1), jnp.int32(-2147483648))
    hi0 = jnp.full((_BLOCK_ROWS, 1), jnp.int32(2147483647))
    lo, _ = lax.fori_loop(0, 32, body, (lo0, hi0))
    ti = jnp.where(lo >= 0, lo, lo ^ jnp.int32(0x7FFFFFFF))
    thres = lax.bitcast_convert_type(ti, jnp.float32)
    o_ref[...] = (x >= thres).astype(jnp.int32)


def kernel(scores, k):
    k_arr = jnp.reshape(jnp.asarray(k, jnp.int32), (1,))
    grid = (_ROWS // _BLOCK_ROWS,)
    return pl.pallas_call(
        _topk_mask_body,
        grid=grid,
        in_specs=[
            pl.BlockSpec(memory_space=pltpu.SMEM),
            pl.BlockSpec((_BLOCK_ROWS, _COLS), lambda i: (i, 0)),
        ],
        out_specs=pl.BlockSpec((_BLOCK_ROWS, _COLS), lambda i: (i, 0)),
        out_shape=jax.ShapeDtypeStruct((_ROWS, _COLS), jnp.int32),
    )(k_arr, scores)
